# Initial kernel scaffold; baseline (speedup 1.0000x reference)
import jax
import jax.numpy as jnp
from jax.experimental import pallas as pl


def kernel(points, queries, feat_prev, W1, b1, W2, b2, Wconv, bconv):
    raise NotImplementedError("write your pallas kernel here")


# speedup vs baseline: 7.2767x; 7.2767x over previous
import functools

import jax
import jax.numpy as jnp
from jax.experimental import pallas as pl
from jax.experimental.pallas import tpu as pltpu
from jax.experimental.pallas import tpu_sc as plsc

KNN = 32
DIV = 4
SHELL = 8
EPS = 1e-5


def _topk_body(q_ref, pt_ref, idx_ref):
    q = q_ref[0]
    p = pt_ref[0]
    d2 = (q[:, 0:1] - p[0:1, :]) ** 2 + (q[:, 1:2] - p[1:2, :]) ** 2
    d2 = d2 + (q[:, 2:3] - p[2:3, :]) ** 2
    m = jnp.maximum(d2, 1e-12)
    dist = m * jax.lax.rsqrt(m)
    mq, n = dist.shape
    iota = jax.lax.broadcasted_iota(jnp.int32, (mq, n), 1)
    work = dist
    cols = []
    for _ in range(KNN):
        m0 = jnp.min(work, axis=1, keepdims=True)
        eq = work == m0
        idx = jnp.min(jnp.where(eq, iota, n), axis=1, keepdims=True)
        cols.append(idx)
        work = jnp.where(eq & (iota == idx), jnp.inf, work)
    idx_ref[0] = jnp.concatenate(cols, axis=1)


def _topk(points_t, queries, mq=256):
    b, m, _ = queries.shape
    n = points_t.shape[2]
    return pl.pallas_call(
        _topk_body,
        grid=(b, m // mq),
        in_specs=[
            pl.BlockSpec((1, mq, 3), lambda bi, i: (bi, i, 0)),
            pl.BlockSpec((1, 3, n), lambda bi, i: (bi, 0, 0)),
        ],
        out_specs=pl.BlockSpec((1, mq, KNN), lambda bi, i: (bi, i, 0)),
        out_shape=jax.ShapeDtypeStruct((b, m, KNN), jnp.int32),
    )(queries, points_t)


def _sc_gather(feat_flat, flat_idx):
    bmk = flat_idx.shape[1]
    fdim = feat_flat.shape[1]
    wf = 128
    mesh = plsc.VectorSubcoreMesh(core_axis_name="c", subcore_axis_name="s")

    @pl.kernel(
        out_type=jax.ShapeDtypeStruct((bmk, fdim), jnp.float32),
        mesh=mesh,
    )
    def k(feat_hbm, idx_hbm, of_hbm):
        def fbody(i_vmem, o_vmem):
            pltpu.sync_copy(feat_hbm.at[i_vmem.at[0]], o_vmem)

        pltpu.emit_pipeline(
            fbody,
            grid=(bmk // wf,),
            in_specs=[pl.BlockSpec((1, wf), lambda i: (0, i))],
            out_specs=[pl.BlockSpec((wf, fdim), lambda i: (i, 0))],
            core_axis_name=("c", "s"),
            dimension_semantics=(pltpu.PARALLEL,),
        )(idx_hbm, of_hbm)

    return k(feat_flat, flat_idx)


def _loc_body(idx_ref, p2d_ref, q_ref, loc_ref, out_ref):
    step = pl.program_id(0)

    @pl.when(step == 0)
    def _():
        out_ref[...] = jnp.zeros_like(out_ref)

    idx = idx_ref[...]
    nq = idx.shape[0]
    lo = idx & 127
    hi = idx >> 7
    oh_lo = (jax.lax.broadcasted_iota(jnp.int32, (nq, KNN, 128), 2)
             == lo[:, :, None]).astype(jnp.float32)
    oh_hi = (jax.lax.broadcasted_iota(jnp.int32, (nq, KNN, 32), 2)
             == hi[:, :, None]).astype(jnp.float32)
    t = jnp.dot(oh_lo.reshape(nq * KNN, 128), p2d_ref[0],
                preferred_element_type=jnp.float32,
                precision=jax.lax.Precision.HIGHEST).reshape(nq, KNN, 96)
    q = q_ref[...]
    chans = []
    for c in range(3):
        sel = jnp.sum(t[:, :, c * 32:(c + 1) * 32] * oh_hi, axis=2)
        chans.append(q[:, c:c + 1] - sel)
    loc = jnp.stack(chans, axis=-1)
    loc_ref[...] = loc
    s1 = jnp.sum(loc, axis=(0, 1)).reshape(1, 3)
    s2 = jnp.sum(loc * loc, axis=(0, 1)).reshape(1, 3)
    out_ref[...] += jnp.concatenate([s1, s2], axis=0)


def _loc_stats(idxs_r, p2d, queries_r, nq=128):
    bm = idxs_r.shape[0]
    m_per_b = bm // p2d.shape[0]
    return pl.pallas_call(
        _loc_body,
        grid=(bm // nq,),
        in_specs=[
            pl.BlockSpec((nq, KNN), lambda i: (i, 0)),
            pl.BlockSpec((1, 128, 96), lambda i: (i // (m_per_b // nq), 0, 0)),
            pl.BlockSpec((nq, 3), lambda i: (i, 0)),
        ],
        out_specs=[
            pl.BlockSpec((nq, KNN, 3), lambda i: (i, 0, 0)),
            pl.BlockSpec((2, 3), lambda i: (0, 0)),
        ],
        out_shape=[
            jax.ShapeDtypeStruct((bm, KNN, 3), jnp.float32),
            jax.ShapeDtypeStruct((2, 3), jnp.float32),
        ],
    )(idxs_r, p2d, queries_r)


def _p2_body(loc_ref, a1_ref, c1_ref, out_ref):
    step = pl.program_id(0)

    @pl.when(step == 0)
    def _():
        out_ref[...] = jnp.zeros_like(out_ref)

    nq = loc_ref.shape[0]
    loc = loc_ref[...].reshape(nq * KNN, 3)
    x1 = jnp.dot(loc, a1_ref[...], preferred_element_type=jnp.float32,
                 precision=jax.lax.Precision.DEFAULT)
    x1 = jnp.maximum(x1 + c1_ref[...], 0.0)
    s1 = jnp.sum(x1, axis=0, keepdims=True)
    s2 = jnp.sum(x1 * x1, axis=0, keepdims=True)
    out_ref[...] += jnp.concatenate([s1, s2], axis=0)


def _p2(loc, a1, c1, nq=1024):
    bm = loc.shape[0]
    return pl.pallas_call(
        _p2_body,
        grid=(bm // nq,),
        in_specs=[
            pl.BlockSpec((nq, KNN, 3), lambda i: (i, 0, 0)),
            pl.BlockSpec((3, 32), lambda i: (0, 0)),
            pl.BlockSpec((1, 32), lambda i: (0, 0)),
        ],
        out_specs=pl.BlockSpec((2, 32), lambda i: (0, 0)),
        out_shape=jax.ShapeDtypeStruct((2, 32), jnp.float32),
    )(loc, a1, c1)


def _p3_body(loc_ref, f_ref, a1_ref, c1_ref, a2_ref, c2_ref,
             xmx_ref, maxf_ref, sx_ref, sf_ref):
    step = pl.program_id(0)

    @pl.when(step == 0)
    def _():
        sx_ref[...] = jnp.zeros_like(sx_ref)
        sf_ref[...] = jnp.zeros_like(sf_ref)

    nq = loc_ref.shape[0]
    loc = loc_ref[...].reshape(nq * KNN, 3)
    x1 = jnp.dot(loc, a1_ref[...], preferred_element_type=jnp.float32,
                 precision=jax.lax.Precision.DEFAULT)
    x1 = jnp.maximum(x1 + c1_ref[...], 0.0)
    x2 = jnp.dot(x1, a2_ref[...], preferred_element_type=jnp.float32,
                 precision=jax.lax.Precision.DEFAULT)
    x2 = jnp.maximum(x2 + c2_ref[...], 0.0)
    x2r = x2.reshape(nq, KNN, 64)
    f = f_ref[...]
    sx = jnp.zeros((2, 64), jnp.float32)
    sf = jnp.zeros((2, 128), jnp.float32)
    for w in range(DIV):
        xw = jnp.max(x2r[:, w * SHELL:(w + 1) * SHELL, :], axis=1)
        fw = jnp.max(f[:, w * SHELL:(w + 1) * SHELL, :], axis=1)
        xmx_ref[w] = xw
        maxf_ref[w] = fw
        sx = sx + jnp.concatenate(
            [jnp.sum(xw, axis=0, keepdims=True),
             jnp.sum(xw * xw, axis=0, keepdims=True)], axis=0)
        sf = sf + jnp.concatenate(
            [jnp.sum(fw, axis=0, keepdims=True),
             jnp.sum(fw * fw, axis=0, keepdims=True)], axis=0)
    sx_ref[...] += sx
    sf_ref[...] += sf


def _p3(loc, feat_g, a1, c1, a2, c2, nq=512):
    bm = loc.shape[0]
    return pl.pallas_call(
        _p3_body,
        grid=(bm // nq,),
        in_specs=[
            pl.BlockSpec((nq, KNN, 3), lambda i: (i, 0, 0)),
            pl.BlockSpec((nq, KNN, 128), lambda i: (i, 0, 0)),
            pl.BlockSpec((3, 32), lambda i: (0, 0)),
            pl.BlockSpec((1, 32), lambda i: (0, 0)),
            pl.BlockSpec((32, 64), lambda i: (0, 0)),
            pl.BlockSpec((1, 64), lambda i: (0, 0)),
        ],
        out_specs=[
            pl.BlockSpec((DIV, nq, 64), lambda i: (0, i, 0)),
            pl.BlockSpec((DIV, nq, 128), lambda i: (0, i, 0)),
            pl.BlockSpec((2, 64), lambda i: (0, 0)),
            pl.BlockSpec((2, 128), lambda i: (0, 0)),
        ],
        out_shape=[
            jax.ShapeDtypeStruct((DIV, bm, 64), jnp.float32),
            jax.ShapeDtypeStruct((DIV, bm, 128), jnp.float32),
            jax.ShapeDtypeStruct((2, 64), jnp.float32),
            jax.ShapeDtypeStruct((2, 128), jnp.float32),
        ],
    )(loc, feat_g, a1, c1, a2, c2)


def _p4_body(xmx_ref, maxf_ref, a3x_ref, a3f_ref, c3_ref, out_ref):
    nq = out_ref.shape[0]
    acc = jnp.broadcast_to(c3_ref[...], (nq, 256))
    for w in range(DIV):
        acc = acc + jnp.dot(xmx_ref[w], a3x_ref[w],
                            preferred_element_type=jnp.float32,
                            precision=jax.lax.Precision.DEFAULT)
        acc = acc + jnp.dot(maxf_ref[w], a3f_ref[w],
                            preferred_element_type=jnp.float32,
                            precision=jax.lax.Precision.DEFAULT)
    out_ref[...] = jnp.maximum(acc, 0.0)


def _p4(xmx, maxf, a3x, a3f, c3, nq=1024):
    bm = xmx.shape[1]
    return pl.pallas_call(
        _p4_body,
        grid=(bm // nq,),
        in_specs=[
            pl.BlockSpec((DIV, nq, 64), lambda i: (0, i, 0)),
            pl.BlockSpec((DIV, nq, 128), lambda i: (0, i, 0)),
            pl.BlockSpec((DIV, 64, 256), lambda i: (0, 0, 0)),
            pl.BlockSpec((DIV, 128, 256), lambda i: (0, 0, 0)),
            pl.BlockSpec((1, 256), lambda i: (0, 0)),
        ],
        out_specs=pl.BlockSpec((nq, 256), lambda i: (i, 0)),
        out_shape=jax.ShapeDtypeStruct((bm, 256), jnp.float32),
    )(xmx, maxf, a3x, a3f, c3)


def kernel(points, queries, feat_prev, W1, b1, W2, b2, Wconv, bconv):
    b, n, _ = points.shape
    m = queries.shape[1]
    bm = b * m
    bmk = bm * KNN

    points_t = jnp.transpose(points, (0, 2, 1))
    idxs = _topk(points_t, queries)

    flat_idx = (idxs + (jnp.arange(b, dtype=jnp.int32) * n)[:, None, None])
    flat_idx = flat_idx.reshape(1, bmk)
    feat_flat = feat_prev.reshape(b * n, 128)

    feat_g = _sc_gather(feat_flat, flat_idx)
    feat_g = feat_g.reshape(bm, KNN, 128)
    queries_r = queries.reshape(bm, 3)
    idxs_r = idxs.reshape(bm, KNN)
    p2d = jnp.transpose(points.reshape(b, 32, 128, 3),
                        (0, 2, 3, 1)).reshape(b, 128, 96)

    loc, s = _loc_stats(idxs_r, p2d, queries_r)
    cnt1 = jnp.float32(bmk)
    mu1 = s[0] / cnt1
    sig1 = jnp.sqrt(jnp.maximum(s[1] / cnt1 - mu1 * mu1, 0.0) + EPS)
    a1 = (W1 / sig1[None, :]).T
    c1 = (b1 - (mu1 / sig1) @ W1.T).reshape(1, 32)

    s = _p2(loc, a1, c1)
    mu2 = s[0] / cnt1
    sig2 = jnp.sqrt(jnp.maximum(s[1] / cnt1 - mu2 * mu2, 0.0) + EPS)
    a2 = (W2 / sig2[None, :]).T
    c2 = (b2 - (mu2 / sig2) @ W2.T).reshape(1, 64)

    xmx, maxf, sx, sf = _p3(loc, feat_g, a1, c1, a2, c2)
    cnt3 = jnp.float32(bm * DIV)
    mu3 = jnp.concatenate([sx[0], sf[0]]) / cnt3
    sig3 = jnp.sqrt(jnp.maximum(
        jnp.concatenate([sx[1], sf[1]]) / cnt3 - mu3 * mu3, 0.0) + EPS)

    wc = Wconv[:, :, 0, :]
    wn = wc / sig3[None, :, None]
    a3x = jnp.transpose(wn[:, :64, :], (2, 1, 0))
    a3f = jnp.transpose(wn[:, 64:, :], (2, 1, 0))
    c3 = (bconv - jnp.einsum('ocw,c->o', wc, mu3 / sig3)).reshape(1, 256)

    out = _p4(xmx, maxf, a3x, a3f, c3)
    return out.reshape(b, m, 256)
